# baseline (speedup 1.0000x reference)
import functools

import jax
import jax.numpy as jnp
from jax import lax
from jax.experimental import pallas as pl
from jax.experimental.pallas import tpu as pltpu
from jax.experimental.pallas import tpu_sc as plsc

N_NODES = 10000
N_EDGES = 320000
D = 128

NC = 2
NS = 16
EDGES_PER_TILE = N_EDGES // (NC * NS)
CHUNK = 128
E_TILE = 10240
N_CHUNKS = E_TILE // CHUNK
NBUF = 2
MAIN_ITERS = N_CHUNKS // (2 * NBUF) - 1
N_PAD = 10240
ROWS_PER_TILE = N_PAD // NS
DEG_W = 128


def _sc_deg_body(dst_hbm, z16_hbm, on16_hbm,
                 deg_out, deg_sh,
                 idxd_v, idxd2_v, s16_v, on_v, si0, si1):
    cid = lax.axis_index("c")
    sid = lax.axis_index("s")
    r0 = sid * ROWS_PER_TILE

    pltpu.sync_copy(z16_hbm, s16_v)
    for j in range(ROWS_PER_TILE // CHUNK):
        pltpu.sync_copy(s16_v, deg_sh.at[pl.ds(r0 + j * CHUNK, CHUNK)])
    pltpu.sync_copy(on16_hbm, on_v)
    plsc.subcore_barrier()

    ebase = (cid * NS + sid) * E_TILE
    bufs = [idxd_v, idxd2_v]
    sems = [si0, si1]

    def issue(i, b):
        e0 = pl.multiple_of(ebase + i * CHUNK, 8)
        pltpu.async_copy(dst_hbm.at[pl.ds(e0, CHUNK)], bufs[b], sems[b])

    def drain(i, b):
        e0 = pl.multiple_of(ebase + i * CHUNK, 8)
        pltpu.make_async_copy(dst_hbm.at[pl.ds(e0, CHUNK)], bufs[b],
                              sems[b]).wait()

    issue(0, 0)
    issue(1, 1)

    def step(it, carry):
        for b in range(2):
            i = 2 * it + b
            drain(i, b)
            pltpu.sync_copy(on_v, deg_sh.at[bufs[b]], add=True)
            issue(i + 2, b)
        return carry

    lax.fori_loop(0, N_CHUNKS // 2 - 1, step, 0)

    for b in range(2):
        drain(N_CHUNKS - 2 + b, b)
        pltpu.sync_copy(on_v, deg_sh.at[bufs[b]], add=True)

    plsc.subcore_barrier()
    for j in range(ROWS_PER_TILE // CHUNK):
        rr = r0 + j * CHUNK
        pltpu.sync_copy(deg_sh.at[pl.ds(rr, CHUNK)], s16_v)
        pltpu.sync_copy(s16_v, deg_out.at[cid, pl.ds(rr, CHUNK)])


def _sc_body(y_hbm, src_hbm, dst_hbm, zf_hbm,
             agg_out, agg_sh,
             sa0, sa1, sb0, sb1,
             da0, da1, db0, db1,
             r0_v, r1_v,
             g0, g1, i0, i1):
    S = [[sa0, sa1], [sb0, sb1]]
    Dd = [[da0, da1], [db0, db1]]
    rows = [r0_v, r1_v]
    gsem = [g0, g1]
    isem = [i0, i1]
    cid = lax.axis_index("c")
    sid = lax.axis_index("s")
    r0 = sid * ROWS_PER_TILE

    pltpu.sync_copy(zf_hbm.at[pl.ds(r0, CHUNK)], r0_v)
    for j in range(ROWS_PER_TILE // CHUNK):
        pltpu.sync_copy(r0_v, agg_sh.at[pl.ds(r0 + j * CHUNK, CHUNK)])
    plsc.subcore_barrier()

    ebase = (cid * NS + sid) * E_TILE

    def idx_issue(i, b, p):
        e0 = pl.multiple_of(ebase + i * CHUNK, 8)
        pltpu.async_copy(src_hbm.at[pl.ds(e0, CHUNK)], S[p][b], isem[b])
        pltpu.async_copy(dst_hbm.at[pl.ds(e0, CHUNK)], Dd[p][b], isem[b])

    def idx_drain(i, b, p):
        e0 = pl.multiple_of(ebase + i * CHUNK, 8)
        pltpu.make_async_copy(src_hbm.at[pl.ds(e0, CHUNK)], S[p][b],
                              isem[b]).wait()
        pltpu.make_async_copy(dst_hbm.at[pl.ds(e0, CHUNK)], Dd[p][b],
                              isem[b]).wait()

    def gather_start(b, p):
        pltpu.async_copy(y_hbm.at[S[p][b]], rows[b], gsem[b])

    def gather_drain_scatter(b, p):
        pltpu.make_async_copy(y_hbm.at[S[p][b]], rows[b], gsem[b]).wait()
        pltpu.sync_copy(rows[b], agg_sh.at[Dd[p][b]], add=True)

    for b in range(NBUF):
        e0 = pl.multiple_of(ebase + b * CHUNK, 8)
        pltpu.sync_copy(src_hbm.at[pl.ds(e0, CHUNK)], S[0][b])
        pltpu.sync_copy(dst_hbm.at[pl.ds(e0, CHUNK)], Dd[0][b])
        gather_start(b, 0)
    for b in range(NBUF):
        idx_issue(NBUF + b, b, 1)

    def cycle(g, p):
        for b in range(NBUF):
            gather_drain_scatter(b, p)
            idx_drain(g + NBUF + b, b, 1 - p)
            gather_start(b, 1 - p)
            idx_issue(g + 2 * NBUF + b, b, p)

    def step(it, carry):
        cycle(2 * NBUF * it, 0)
        cycle(2 * NBUF * it + NBUF, 1)
        return carry

    lax.fori_loop(0, MAIN_ITERS, step, 0)

    for b in range(NBUF):
        gather_drain_scatter(b, 0)
    for b in range(NBUF):
        idx_drain(N_CHUNKS - NBUF + b, b, 1)
    for b in range(NBUF):
        gather_start(b, 1)
    for b in range(NBUF):
        gather_drain_scatter(b, 1)

    plsc.subcore_barrier()
    for j in range(ROWS_PER_TILE // CHUNK):
        rr = r0 + j * CHUNK
        pltpu.sync_copy(agg_sh.at[pl.ds(rr, CHUNK)], r0_v)
        pltpu.sync_copy(r0_v, agg_out.at[cid, pl.ds(rr, CHUNK)])


_sc_mesh = plsc.VectorSubcoreMesh(core_axis_name="c", subcore_axis_name="s",
                                  num_cores=NC, num_subcores=NS)

_sc_deg = pl.kernel(
    _sc_deg_body,
    out_type=jax.ShapeDtypeStruct((NC, N_PAD, DEG_W), jnp.float32),
    mesh=_sc_mesh,
    scratch_types=[
        pltpu.VMEM_SHARED((N_PAD, DEG_W), jnp.float32),
        pltpu.VMEM((CHUNK,), jnp.int32),
        pltpu.VMEM((CHUNK,), jnp.int32),
        pltpu.VMEM((CHUNK, DEG_W), jnp.float32),
        pltpu.VMEM((CHUNK, DEG_W), jnp.float32),
        pltpu.SemaphoreType.DMA,
        pltpu.SemaphoreType.DMA,
    ],
)

_sc_agg = pl.kernel(
    _sc_body,
    out_type=jax.ShapeDtypeStruct((NC, N_PAD, D), jnp.float32),
    mesh=_sc_mesh,
    scratch_types=[pltpu.VMEM_SHARED((N_PAD, D), jnp.float32)]
      + [pltpu.VMEM((CHUNK,), jnp.int32)] * (4 * NBUF)
      + [pltpu.VMEM((CHUNK, D), jnp.float32)] * NBUF
      + [pltpu.SemaphoreType.DMA] * (2 * NBUF),
)


_BLK = 1000
_GRID = N_NODES // _BLK

_DN = (((1,), (1,)), ((), ()))


def _mm2_body(x_ref, wn_ref, ws_ref, y_ref, s_ref):
    x = x_ref[...]
    y_ref[...] = lax.dot_general(x, wn_ref[...], _DN,
                                 preferred_element_type=jnp.float32)
    s_ref[...] = lax.dot_general(x, ws_ref[...], _DN,
                                 preferred_element_type=jnp.float32)


_mm2 = pl.pallas_call(
    _mm2_body,
    grid=(_GRID,),
    in_specs=[
        pl.BlockSpec((_BLK, D), lambda i: (i, 0)),
        pl.BlockSpec((D, D), lambda i: (0, 0)),
        pl.BlockSpec((D, D), lambda i: (0, 0)),
    ],
    out_specs=(
        pl.BlockSpec((_BLK, D), lambda i: (i, 0)),
        pl.BlockSpec((_BLK, D), lambda i: (i, 0)),
    ),
    out_shape=(
        jax.ShapeDtypeStruct((N_NODES, D), jnp.float32),
        jax.ShapeDtypeStruct((N_NODES, D), jnp.float32),
    ),
)


def _mid_body(s1_ref, agg_ref, deg_ref, wn_ref, ws_ref, b1_ref, y2_ref, s2_ref):
    agg = agg_ref[0] + agg_ref[1]
    deg = deg_ref[0, :, 0:1] + deg_ref[1, :, 0:1]
    inv = 1.0 / jnp.maximum(deg, 1.0)
    h = jnp.maximum(s1_ref[...] + agg * inv + b1_ref[...], 0.0)
    y2_ref[...] = lax.dot_general(h, wn_ref[...], _DN,
                                  preferred_element_type=jnp.float32)
    s2_ref[...] = lax.dot_general(h, ws_ref[...], _DN,
                                  preferred_element_type=jnp.float32)


_mid = pl.pallas_call(
    _mid_body,
    grid=(_GRID,),
    in_specs=[
        pl.BlockSpec((_BLK, D), lambda i: (i, 0)),
        pl.BlockSpec((NC, _BLK, D), lambda i: (0, i, 0)),
        pl.BlockSpec((NC, _BLK, DEG_W), lambda i: (0, i, 0)),
        pl.BlockSpec((D, D), lambda i: (0, 0)),
        pl.BlockSpec((D, D), lambda i: (0, 0)),
        pl.BlockSpec((1, D), lambda i: (0, 0)),
    ],
    out_specs=(
        pl.BlockSpec((_BLK, D), lambda i: (i, 0)),
        pl.BlockSpec((_BLK, D), lambda i: (i, 0)),
    ),
    out_shape=(
        jax.ShapeDtypeStruct((N_NODES, D), jnp.float32),
        jax.ShapeDtypeStruct((N_NODES, D), jnp.float32),
    ),
)


def _fin_body(s2_ref, agg_ref, deg_ref, b2_ref, out_ref):
    agg = agg_ref[0] + agg_ref[1]
    deg = deg_ref[0, :, 0:1] + deg_ref[1, :, 0:1]
    inv = 1.0 / jnp.maximum(deg, 1.0)
    out_ref[...] = s2_ref[...] + agg * inv + b2_ref[...]


_fin = pl.pallas_call(
    _fin_body,
    grid=(_GRID,),
    in_specs=[
        pl.BlockSpec((_BLK, D), lambda i: (i, 0)),
        pl.BlockSpec((NC, _BLK, D), lambda i: (0, i, 0)),
        pl.BlockSpec((NC, _BLK, DEG_W), lambda i: (0, i, 0)),
        pl.BlockSpec((1, D), lambda i: (0, 0)),
    ],
    out_specs=pl.BlockSpec((_BLK, D), lambda i: (i, 0)),
    out_shape=jax.ShapeDtypeStruct((N_NODES, D), jnp.float32),
)


def kernel(in_feat, edge_index, W1_self, W1_neigh, b1, W2_self, W2_neigh, b2):
    epad = ((0, 0), (0, E_TILE - EDGES_PER_TILE))
    src = jnp.pad(edge_index[0].astype(jnp.int32).reshape(NC * NS, -1),
                  epad).reshape(-1)
    dst = jnp.pad(edge_index[1].astype(jnp.int32).reshape(NC * NS, -1),
                  epad, constant_values=N_NODES).reshape(-1)
    zf = jnp.zeros((N_PAD, D), jnp.float32)
    z16 = jnp.zeros((CHUNK, DEG_W), jnp.float32)
    on16 = jnp.ones((CHUNK, DEG_W), jnp.float32)

    degp = _sc_deg(dst, z16, on16)
    y1, s1 = _mm2(in_feat, W1_neigh, W1_self)
    agg1 = _sc_agg(y1, src, dst, zf)
    y2, s2 = _mid(s1, agg1, degp, W2_neigh, W2_self, b1.reshape(1, D))
    agg2 = _sc_agg(y2, src, dst, zf)
    return _fin(s2, agg2, degp, b2.reshape(1, D))


# speedup vs baseline: 2.8094x; 2.8094x over previous
import functools

import jax
import jax.numpy as jnp
from jax import lax
from jax.experimental import pallas as pl
from jax.experimental.pallas import tpu as pltpu
from jax.experimental.pallas import tpu_sc as plsc

N_NODES = 10000
N_EDGES = 320000
D = 128

NC = 2
NS = 16
EDGES_PER_TILE = N_EDGES // (NC * NS)
CHUNK = 80
E_TILE = EDGES_PER_TILE
N_CHUNKS = E_TILE // CHUNK
NBUF = 4
MAIN_ITERS = 15
N_PAD = 10240
ROWS_PER_TILE = N_PAD // NS
DEG_W = 128
E_PAD = N_EDGES + 256


def _sc_deg_body(dst_hbm, z16_hbm, on16_hbm,
                 deg_out, deg_sh,
                 idxd_v, idxd2_v, s16_v, on_v, si0, si1):
    cid = lax.axis_index("c")
    sid = lax.axis_index("s")
    r0 = sid * ROWS_PER_TILE

    pltpu.sync_copy(z16_hbm, s16_v)
    for j in range(ROWS_PER_TILE // CHUNK):
        pltpu.sync_copy(s16_v, deg_sh.at[pl.ds(r0 + j * CHUNK, CHUNK)])
    pltpu.sync_copy(on16_hbm, on_v)
    plsc.subcore_barrier()

    ebase = (cid * NS + sid) * E_TILE
    bufs = [idxd_v, idxd2_v]
    sems = [si0, si1]

    def issue(i, b):
        e0 = pl.multiple_of(ebase + i * CHUNK, 8)
        pltpu.async_copy(dst_hbm.at[pl.ds(e0, CHUNK)], bufs[b], sems[b])

    def drain(i, b):
        e0 = pl.multiple_of(ebase + i * CHUNK, 8)
        pltpu.make_async_copy(dst_hbm.at[pl.ds(e0, CHUNK)], bufs[b],
                              sems[b]).wait()

    issue(0, 0)
    issue(1, 1)

    def step(it, carry):
        for b in range(2):
            i = 2 * it + b
            drain(i, b)
            pltpu.sync_copy(on_v, deg_sh.at[bufs[b]], add=True)
            issue(i + 2, b)
        return carry

    lax.fori_loop(0, (N_CHUNKS - 1) // 2, step, 0)

    drain(N_CHUNKS - 1, 0)
    pltpu.sync_copy(on_v, deg_sh.at[bufs[0]], add=True)
    drain(N_CHUNKS, 1)

    plsc.subcore_barrier()
    for j in range(ROWS_PER_TILE // CHUNK):
        rr = r0 + j * CHUNK
        pltpu.sync_copy(deg_sh.at[pl.ds(rr, CHUNK)], s16_v)
        pltpu.sync_copy(s16_v, deg_out.at[cid, pl.ds(rr, CHUNK)])


def _sc_body(y_hbm, src_hbm, dst_hbm, zf_hbm,
             agg_out, agg_sh,
             sa0, sa1, sa2, sa3, sb0, sb1, sb2, sb3,
             da0, da1, da2, da3, db0, db1, db2, db3,
             r0_v, r1_v, r2_v, r3_v,
             g0, g1, g2, g3, i0, i1, i2, i3):
    S = [[sa0, sa1, sa2, sa3], [sb0, sb1, sb2, sb3]]
    Dd = [[da0, da1, da2, da3], [db0, db1, db2, db3]]
    rows = [r0_v, r1_v, r2_v, r3_v]
    gsem = [g0, g1, g2, g3]
    isem = [i0, i1, i2, i3]
    cid = lax.axis_index("c")
    sid = lax.axis_index("s")
    r0 = sid * ROWS_PER_TILE

    pltpu.sync_copy(zf_hbm.at[pl.ds(r0, CHUNK)], r0_v)
    for j in range(ROWS_PER_TILE // CHUNK):
        pltpu.sync_copy(r0_v, agg_sh.at[pl.ds(r0 + j * CHUNK, CHUNK)])
    plsc.subcore_barrier()

    ebase = (cid * NS + sid) * E_TILE

    def idx_issue(i, b, p):
        e0 = pl.multiple_of(ebase + i * CHUNK, 8)
        pltpu.async_copy(src_hbm.at[pl.ds(e0, CHUNK)], S[p][b], isem[b])
        pltpu.async_copy(dst_hbm.at[pl.ds(e0, CHUNK)], Dd[p][b], isem[b])

    def idx_drain(i, b, p):
        e0 = pl.multiple_of(ebase + i * CHUNK, 8)
        pltpu.make_async_copy(src_hbm.at[pl.ds(e0, CHUNK)], S[p][b],
                              isem[b]).wait()
        pltpu.make_async_copy(dst_hbm.at[pl.ds(e0, CHUNK)], Dd[p][b],
                              isem[b]).wait()

    def gather_start(b, p):
        pltpu.async_copy(y_hbm.at[S[p][b]], rows[b], gsem[b])

    def gather_drain_scatter(b, p):
        pltpu.make_async_copy(y_hbm.at[S[p][b]], rows[b], gsem[b]).wait()
        pltpu.sync_copy(rows[b], agg_sh.at[Dd[p][b]], add=True)

    for b in range(NBUF):
        e0 = pl.multiple_of(ebase + b * CHUNK, 8)
        pltpu.sync_copy(src_hbm.at[pl.ds(e0, CHUNK)], S[0][b])
        pltpu.sync_copy(dst_hbm.at[pl.ds(e0, CHUNK)], Dd[0][b])
        gather_start(b, 0)
    for b in range(NBUF):
        idx_issue(NBUF + b, b, 1)

    def cycle(g, p):
        for b in range(NBUF):
            gather_drain_scatter(b, p)
            idx_drain(g + NBUF + b, b, 1 - p)
            gather_start(b, 1 - p)
            idx_issue(g + 2 * NBUF + b, b, p)

    def step(it, carry):
        cycle(2 * NBUF * it, 0)
        cycle(2 * NBUF * it + NBUF, 1)
        return carry

    lax.fori_loop(0, MAIN_ITERS, step, 0)

    for b in range(NBUF):
        gather_drain_scatter(b, 0)
    for b in range(NBUF):
        idx_drain(30 * NBUF + b + NBUF, b, 1)

    gather_start(0, 1)
    gather_drain_scatter(0, 1)

    plsc.subcore_barrier()
    for j in range(ROWS_PER_TILE // CHUNK):
        rr = r0 + j * CHUNK
        pltpu.sync_copy(agg_sh.at[pl.ds(rr, CHUNK)], r0_v)
        pltpu.sync_copy(r0_v, agg_out.at[cid, pl.ds(rr, CHUNK)])


_sc_mesh = plsc.VectorSubcoreMesh(core_axis_name="c", subcore_axis_name="s",
                                  num_cores=NC, num_subcores=NS)

_sc_deg = pl.kernel(
    _sc_deg_body,
    out_type=jax.ShapeDtypeStruct((NC, N_PAD, DEG_W), jnp.float32),
    mesh=_sc_mesh,
    scratch_types=[
        pltpu.VMEM_SHARED((N_PAD, DEG_W), jnp.float32),
        pltpu.VMEM((CHUNK,), jnp.int32),
        pltpu.VMEM((CHUNK,), jnp.int32),
        pltpu.VMEM((CHUNK, DEG_W), jnp.float32),
        pltpu.VMEM((CHUNK, DEG_W), jnp.float32),
        pltpu.SemaphoreType.DMA,
        pltpu.SemaphoreType.DMA,
    ],
)

_sc_agg = pl.kernel(
    _sc_body,
    out_type=jax.ShapeDtypeStruct((NC, N_PAD, D), jnp.float32),
    mesh=_sc_mesh,
    scratch_types=[pltpu.VMEM_SHARED((N_PAD, D), jnp.float32)]
      + [pltpu.VMEM((CHUNK,), jnp.int32)] * (4 * NBUF)
      + [pltpu.VMEM((CHUNK, D), jnp.float32)] * NBUF
      + [pltpu.SemaphoreType.DMA] * (2 * NBUF),
)


_BLK = 1000
_GRID = N_NODES // _BLK

_DN = (((1,), (1,)), ((), ()))


def _mm2_body(x_ref, wn_ref, ws_ref, y_ref, s_ref):
    x = x_ref[...]
    y_ref[...] = lax.dot_general(x, wn_ref[...], _DN,
                                 preferred_element_type=jnp.float32)
    s_ref[...] = lax.dot_general(x, ws_ref[...], _DN,
                                 preferred_element_type=jnp.float32)


_mm2 = pl.pallas_call(
    _mm2_body,
    grid=(_GRID,),
    in_specs=[
        pl.BlockSpec((_BLK, D), lambda i: (i, 0)),
        pl.BlockSpec((D, D), lambda i: (0, 0)),
        pl.BlockSpec((D, D), lambda i: (0, 0)),
    ],
    out_specs=(
        pl.BlockSpec((_BLK, D), lambda i: (i, 0)),
        pl.BlockSpec((_BLK, D), lambda i: (i, 0)),
    ),
    out_shape=(
        jax.ShapeDtypeStruct((N_NODES, D), jnp.float32),
        jax.ShapeDtypeStruct((N_NODES, D), jnp.float32),
    ),
)


def _mid_body(s1_ref, agg_ref, deg_ref, wn_ref, ws_ref, b1_ref, y2_ref, s2_ref):
    agg = agg_ref[0] + agg_ref[1]
    deg = deg_ref[0, :, 0:1] + deg_ref[1, :, 0:1]
    inv = 1.0 / jnp.maximum(deg, 1.0)
    h = jnp.maximum(s1_ref[...] + agg * inv + b1_ref[...], 0.0)
    y2_ref[...] = lax.dot_general(h, wn_ref[...], _DN,
                                  preferred_element_type=jnp.float32)
    s2_ref[...] = lax.dot_general(h, ws_ref[...], _DN,
                                  preferred_element_type=jnp.float32)


_mid = pl.pallas_call(
    _mid_body,
    grid=(_GRID,),
    in_specs=[
        pl.BlockSpec((_BLK, D), lambda i: (i, 0)),
        pl.BlockSpec((NC, _BLK, D), lambda i: (0, i, 0)),
        pl.BlockSpec((NC, _BLK, DEG_W), lambda i: (0, i, 0)),
        pl.BlockSpec((D, D), lambda i: (0, 0)),
        pl.BlockSpec((D, D), lambda i: (0, 0)),
        pl.BlockSpec((1, D), lambda i: (0, 0)),
    ],
    out_specs=(
        pl.BlockSpec((_BLK, D), lambda i: (i, 0)),
        pl.BlockSpec((_BLK, D), lambda i: (i, 0)),
    ),
    out_shape=(
        jax.ShapeDtypeStruct((N_NODES, D), jnp.float32),
        jax.ShapeDtypeStruct((N_NODES, D), jnp.float32),
    ),
)


def _fin_body(s2_ref, agg_ref, deg_ref, b2_ref, out_ref):
    agg = agg_ref[0] + agg_ref[1]
    deg = deg_ref[0, :, 0:1] + deg_ref[1, :, 0:1]
    inv = 1.0 / jnp.maximum(deg, 1.0)
    out_ref[...] = s2_ref[...] + agg * inv + b2_ref[...]


_fin = pl.pallas_call(
    _fin_body,
    grid=(_GRID,),
    in_specs=[
        pl.BlockSpec((_BLK, D), lambda i: (i, 0)),
        pl.BlockSpec((NC, _BLK, D), lambda i: (0, i, 0)),
        pl.BlockSpec((NC, _BLK, DEG_W), lambda i: (0, i, 0)),
        pl.BlockSpec((1, D), lambda i: (0, 0)),
    ],
    out_specs=pl.BlockSpec((_BLK, D), lambda i: (i, 0)),
    out_shape=jax.ShapeDtypeStruct((N_NODES, D), jnp.float32),
)


def kernel(in_feat, edge_index, W1_self, W1_neigh, b1, W2_self, W2_neigh, b2):
    pad = jnp.zeros((E_PAD - N_EDGES,), jnp.int32)
    src = jnp.concatenate([edge_index[0].astype(jnp.int32), pad])
    dst = jnp.concatenate([edge_index[1].astype(jnp.int32), pad])
    zf = jnp.zeros((N_PAD, D), jnp.float32)
    z16 = jnp.zeros((CHUNK, DEG_W), jnp.float32)
    on16 = jnp.ones((CHUNK, DEG_W), jnp.float32)

    degp = _sc_deg(dst, z16, on16)
    y1, s1 = _mm2(in_feat, W1_neigh, W1_self)
    agg1 = _sc_agg(y1, src, dst, zf)
    y2, s2 = _mid(s1, agg1, degp, W2_neigh, W2_self, b1.reshape(1, D))
    agg2 = _sc_agg(y2, src, dst, zf)
    return _fin(s2, agg2, degp, b2.reshape(1, D))


# speedup vs baseline: 2.8632x; 1.0192x over previous
import functools

import jax
import jax.numpy as jnp
from jax import lax
from jax.experimental import pallas as pl
from jax.experimental.pallas import tpu as pltpu
from jax.experimental.pallas import tpu_sc as plsc

N_NODES = 10000
N_EDGES = 320000
D = 128

NC = 2
NS = 16
EDGES_PER_TILE = N_EDGES // (NC * NS)
CHUNK = 80
E_TILE = EDGES_PER_TILE
N_CHUNKS = E_TILE // CHUNK
NBUF = 4
MAIN_ITERS = 15
N_PAD = 10240
ROWS_PER_TILE = N_PAD // NS
DEG_W = 128
E_PAD = N_EDGES + 256


def _sc_deg_body(dst_hbm, z16_hbm, on16_hbm,
                 deg_out, deg_sh,
                 idxd_v, idxd2_v, s16_v, on_v, si0, si1):
    cid = lax.axis_index("c")
    sid = lax.axis_index("s")
    r0 = sid * ROWS_PER_TILE

    pltpu.sync_copy(z16_hbm, s16_v)
    for j in range(ROWS_PER_TILE // CHUNK):
        pltpu.sync_copy(s16_v, deg_sh.at[pl.ds(r0 + j * CHUNK, CHUNK)])
    pltpu.sync_copy(on16_hbm, on_v)
    plsc.subcore_barrier()

    ebase = (cid * NS + sid) * E_TILE
    bufs = [idxd_v, idxd2_v]
    sems = [si0, si1]

    def issue(i, b):
        e0 = pl.multiple_of(ebase + i * CHUNK, 8)
        pltpu.async_copy(dst_hbm.at[pl.ds(e0, CHUNK)], bufs[b], sems[b])

    def drain(i, b):
        e0 = pl.multiple_of(ebase + i * CHUNK, 8)
        pltpu.make_async_copy(dst_hbm.at[pl.ds(e0, CHUNK)], bufs[b],
                              sems[b]).wait()

    issue(0, 0)
    issue(1, 1)

    def step(it, carry):
        for b in range(2):
            i = 2 * it + b
            drain(i, b)
            pltpu.sync_copy(on_v, deg_sh.at[bufs[b]], add=True)
            issue(i + 2, b)
        return carry

    lax.fori_loop(0, (N_CHUNKS - 1) // 2, step, 0)

    drain(N_CHUNKS - 1, 0)
    pltpu.sync_copy(on_v, deg_sh.at[bufs[0]], add=True)
    drain(N_CHUNKS, 1)

    plsc.subcore_barrier()
    for j in range(ROWS_PER_TILE // CHUNK):
        rr = r0 + j * CHUNK
        pltpu.sync_copy(deg_sh.at[pl.ds(rr, CHUNK)], s16_v)
        pltpu.sync_copy(s16_v, deg_out.at[cid, pl.ds(rr, CHUNK)])


def _sc_body(y_hbm, src_hbm, dst_hbm, zf_hbm,
             agg_out, agg_sh,
             sa0, sa1, sa2, sa3, sb0, sb1, sb2, sb3,
             da0, da1, da2, da3, db0, db1, db2, db3,
             r0_v, r1_v, r2_v, r3_v,
             g0, g1, g2, g3, i0, i1, i2, i3):
    S = [[sa0, sa1, sa2, sa3], [sb0, sb1, sb2, sb3]]
    Dd = [[da0, da1, da2, da3], [db0, db1, db2, db3]]
    rows = [r0_v, r1_v, r2_v, r3_v]
    gsem = [g0, g1, g2, g3]
    isem = [i0, i1, i2, i3]
    cid = lax.axis_index("c")
    sid = lax.axis_index("s")
    r0 = sid * ROWS_PER_TILE

    pltpu.sync_copy(zf_hbm.at[pl.ds(r0, CHUNK)], r0_v)
    for j in range(ROWS_PER_TILE // CHUNK):
        pltpu.sync_copy(r0_v, agg_sh.at[pl.ds(r0 + j * CHUNK, CHUNK)])
    plsc.subcore_barrier()

    ebase = (cid * NS + sid) * E_TILE

    def idx_issue(i, b, p):
        e0 = pl.multiple_of(ebase + i * CHUNK, 8)
        pltpu.async_copy(src_hbm.at[pl.ds(e0, CHUNK)], S[p][b], isem[b])
        pltpu.async_copy(dst_hbm.at[pl.ds(e0, CHUNK)], Dd[p][b], isem[b])

    def idx_drain(i, b, p):
        e0 = pl.multiple_of(ebase + i * CHUNK, 8)
        pltpu.make_async_copy(src_hbm.at[pl.ds(e0, CHUNK)], S[p][b],
                              isem[b]).wait()
        pltpu.make_async_copy(dst_hbm.at[pl.ds(e0, CHUNK)], Dd[p][b],
                              isem[b]).wait()

    def gather_start(b, p):
        pltpu.async_copy(y_hbm.at[S[p][b]], rows[b], gsem[b])

    def gather_drain_scatter(b, p):
        pltpu.make_async_copy(y_hbm.at[S[p][b]], rows[b], gsem[b]).wait()
        pltpu.sync_copy(rows[b], agg_sh.at[Dd[p][b]], add=True)

    for b in range(NBUF):
        e0 = pl.multiple_of(ebase + b * CHUNK, 8)
        pltpu.sync_copy(src_hbm.at[pl.ds(e0, CHUNK)], S[0][b])
        pltpu.sync_copy(dst_hbm.at[pl.ds(e0, CHUNK)], Dd[0][b])
        gather_start(b, 0)
    for b in range(NBUF):
        idx_issue(NBUF + b, b, 1)

    def cycle(g, p):
        for b in range(NBUF):
            gather_drain_scatter(b, p)
            idx_drain(g + NBUF + b, b, 1 - p)
            gather_start(b, 1 - p)
            idx_issue(g + 2 * NBUF + b, b, p)

    def step(it, carry):
        cycle(2 * NBUF * it, 0)
        cycle(2 * NBUF * it + NBUF, 1)
        return carry

    lax.fori_loop(0, MAIN_ITERS, step, 0)

    for b in range(NBUF):
        gather_drain_scatter(b, 0)
    for b in range(NBUF):
        idx_drain(30 * NBUF + b + NBUF, b, 1)

    gather_start(0, 1)
    gather_drain_scatter(0, 1)

    plsc.subcore_barrier()
    for j in range(ROWS_PER_TILE // CHUNK):
        rr = r0 + j * CHUNK
        pltpu.sync_copy(agg_sh.at[pl.ds(rr, CHUNK)], r0_v)
        pltpu.sync_copy(r0_v, agg_out.at[cid, pl.ds(rr, CHUNK)])


_sc_mesh = plsc.VectorSubcoreMesh(core_axis_name="c", subcore_axis_name="s",
                                  num_cores=NC, num_subcores=NS)

_sc_deg = pl.kernel(
    _sc_deg_body,
    out_type=jax.ShapeDtypeStruct((NC, N_PAD, DEG_W), jnp.float32),
    mesh=_sc_mesh,
    scratch_types=[
        pltpu.VMEM_SHARED((N_PAD, DEG_W), jnp.float32),
        pltpu.VMEM((CHUNK,), jnp.int32),
        pltpu.VMEM((CHUNK,), jnp.int32),
        pltpu.VMEM((CHUNK, DEG_W), jnp.float32),
        pltpu.VMEM((CHUNK, DEG_W), jnp.float32),
        pltpu.SemaphoreType.DMA,
        pltpu.SemaphoreType.DMA,
    ],
)

_sc_agg = pl.kernel(
    _sc_body,
    out_type=jax.ShapeDtypeStruct((NC, N_PAD, D), jnp.float32),
    mesh=_sc_mesh,
    scratch_types=[pltpu.VMEM_SHARED((N_PAD, D), jnp.float32)]
      + [pltpu.VMEM((CHUNK,), jnp.int32)] * (4 * NBUF)
      + [pltpu.VMEM((CHUNK, D), jnp.float32)] * NBUF
      + [pltpu.SemaphoreType.DMA] * (2 * NBUF),
)


_BLK = 2000
_GRID = N_NODES // _BLK

_DN = (((1,), (1,)), ((), ()))


def _mm2_body(x_ref, wn_ref, ws_ref, y_ref, s_ref):
    x = x_ref[...]
    y_ref[...] = lax.dot_general(x, wn_ref[...], _DN,
                                 preferred_element_type=jnp.float32)
    s_ref[...] = lax.dot_general(x, ws_ref[...], _DN,
                                 preferred_element_type=jnp.float32)


_mm2 = pl.pallas_call(
    _mm2_body,
    grid=(_GRID,),
    in_specs=[
        pl.BlockSpec((_BLK, D), lambda i: (i, 0)),
        pl.BlockSpec((D, D), lambda i: (0, 0)),
        pl.BlockSpec((D, D), lambda i: (0, 0)),
    ],
    out_specs=(
        pl.BlockSpec((_BLK, D), lambda i: (i, 0)),
        pl.BlockSpec((_BLK, D), lambda i: (i, 0)),
    ),
    out_shape=(
        jax.ShapeDtypeStruct((N_NODES, D), jnp.float32),
        jax.ShapeDtypeStruct((N_NODES, D), jnp.float32),
    ),
)


def _mid_body(s1_ref, agg_ref, deg_ref, wn_ref, ws_ref, b1_ref, y2_ref, s2_ref):
    agg = agg_ref[0] + agg_ref[1]
    deg = deg_ref[0, :, 0:1] + deg_ref[1, :, 0:1]
    inv = 1.0 / jnp.maximum(deg, 1.0)
    h = jnp.maximum(s1_ref[...] + agg * inv + b1_ref[...], 0.0)
    y2_ref[...] = lax.dot_general(h, wn_ref[...], _DN,
                                  preferred_element_type=jnp.float32)
    s2_ref[...] = lax.dot_general(h, ws_ref[...], _DN,
                                  preferred_element_type=jnp.float32)


_mid = pl.pallas_call(
    _mid_body,
    grid=(_GRID,),
    in_specs=[
        pl.BlockSpec((_BLK, D), lambda i: (i, 0)),
        pl.BlockSpec((NC, _BLK, D), lambda i: (0, i, 0)),
        pl.BlockSpec((NC, _BLK, DEG_W), lambda i: (0, i, 0)),
        pl.BlockSpec((D, D), lambda i: (0, 0)),
        pl.BlockSpec((D, D), lambda i: (0, 0)),
        pl.BlockSpec((1, D), lambda i: (0, 0)),
    ],
    out_specs=(
        pl.BlockSpec((_BLK, D), lambda i: (i, 0)),
        pl.BlockSpec((_BLK, D), lambda i: (i, 0)),
    ),
    out_shape=(
        jax.ShapeDtypeStruct((N_NODES, D), jnp.float32),
        jax.ShapeDtypeStruct((N_NODES, D), jnp.float32),
    ),
)


def _fin_body(s2_ref, agg_ref, deg_ref, b2_ref, out_ref):
    agg = agg_ref[0] + agg_ref[1]
    deg = deg_ref[0, :, 0:1] + deg_ref[1, :, 0:1]
    inv = 1.0 / jnp.maximum(deg, 1.0)
    out_ref[...] = s2_ref[...] + agg * inv + b2_ref[...]


_fin = pl.pallas_call(
    _fin_body,
    grid=(_GRID,),
    in_specs=[
        pl.BlockSpec((_BLK, D), lambda i: (i, 0)),
        pl.BlockSpec((NC, _BLK, D), lambda i: (0, i, 0)),
        pl.BlockSpec((NC, _BLK, DEG_W), lambda i: (0, i, 0)),
        pl.BlockSpec((1, D), lambda i: (0, 0)),
    ],
    out_specs=pl.BlockSpec((_BLK, D), lambda i: (i, 0)),
    out_shape=jax.ShapeDtypeStruct((N_NODES, D), jnp.float32),
)


def kernel(in_feat, edge_index, W1_self, W1_neigh, b1, W2_self, W2_neigh, b2):
    pad = jnp.zeros((E_PAD - N_EDGES,), jnp.int32)
    src = jnp.concatenate([edge_index[0].astype(jnp.int32), pad])
    dst = jnp.concatenate([edge_index[1].astype(jnp.int32), pad])
    zf = jnp.zeros((N_PAD, D), jnp.float32)
    z16 = jnp.zeros((CHUNK, DEG_W), jnp.float32)
    on16 = jnp.ones((CHUNK, DEG_W), jnp.float32)

    degp = _sc_deg(dst, z16, on16)
    y1, s1 = _mm2(in_feat, W1_neigh, W1_self)
    agg1 = _sc_agg(y1, src, dst, zf)
    y2, s2 = _mid(s1, agg1, degp, W2_neigh, W2_self, b1.reshape(1, D))
    agg2 = _sc_agg(y2, src, dst, zf)
    return _fin(s2, agg2, degp, b2.reshape(1, D))
